# baseline (speedup 1.0000x reference)
import functools

import jax
import jax.numpy as jnp
from jax import lax
from jax.experimental import pallas as pl
from jax.experimental.pallas import tpu as pltpu
from jax.experimental.pallas import tpu_sc as plsc

B = 16384
D = 32
PACK = 4
ROWS = 1000000 // PACK
PD = PACK * D
HIDDEN = 64
NC = 2
NS = 16
NW = NC * NS
BPW = B // NW
CHUNK = 128
NCH = BPW // CHUNK

MB = 2048


def _sc_gather(uidx2d, didx2d, utab, dtab):
    mesh = plsc.VectorSubcoreMesh(core_axis_name="c", subcore_axis_name="s")

    @functools.partial(
        pl.kernel,
        mesh=mesh,
        out_type=[
            jax.ShapeDtypeStruct((B, PD), jnp.float32),
            jax.ShapeDtypeStruct((B, PD), jnp.float32),
        ],
        scratch_types=[
            pltpu.VMEM((NCH, CHUNK), jnp.int32),
            pltpu.VMEM((NCH, CHUNK), jnp.int32),
            pltpu.VMEM((2, CHUNK, PD), jnp.float32),
            pltpu.VMEM((2, CHUNK, PD), jnp.float32),
            pltpu.SemaphoreType.DMA,
            pltpu.SemaphoreType.DMA,
            pltpu.SemaphoreType.DMA,
            pltpu.SemaphoreType.DMA,
        ],
    )
    def gather_kernel(uidx_hbm, didx_hbm, utab_hbm, dtab_hbm,
                      u_out, d_out,
                      uidx_v, didx_v, ubuf_v, dbuf_v, gsem0, gsem1,
                      osem0, osem1):
        wid = lax.axis_index("s") * NC + lax.axis_index("c")
        pltpu.sync_copy(uidx_hbm.at[pl.ds(wid * NCH, NCH)], uidx_v)
        pltpu.sync_copy(didx_hbm.at[pl.ds(wid * NCH, NCH)], didx_v)
        base = wid * BPW

        gsems = (gsem0, gsem1)
        osems = (osem0, osem1)

        def chunk_plan(idx_v, tab_hbm, buf_v, out_hbm):
            gathers, outs = [], []
            for j in range(NCH):
                gathers.append(lambda j=j: pltpu.async_copy(
                    tab_hbm.at[idx_v.at[j]], buf_v.at[j % 2], gsems[j % 2]))
                outs.append(lambda j=j: pltpu.async_copy(
                    buf_v.at[j % 2],
                    out_hbm.at[pl.ds(base + j * CHUNK, CHUNK)], osems[j % 2]))
            return gathers, outs

        ug, uo = chunk_plan(uidx_v, utab_hbm, ubuf_v, u_out)
        dg, do = chunk_plan(didx_v, dtab_hbm, dbuf_v, d_out)
        for g, o in ((ug, uo), (dg, do)):
            gc = [None] * NCH
            oc = [None] * NCH
            gc[0] = g[0]()
            gc[1] = g[1]()
            for j in range(NCH):
                gc[j].wait()
                oc[j] = o[j]()
                if j + 2 < NCH:
                    oc[j].wait()
                    gc[j + 2] = g[j + 2]()
            for j in range(NCH - 2, NCH):
                oc[j].wait()

    return gather_kernel(uidx2d, didx2d, utab, dtab)


def _mlp_body(u128_ref, d128_ref, su_ref, sd_ref, img_ref,
              w1u_ref, w1d_ref, w1i_ref, b1_ref, w2t_ref, b2_ref, out_ref):
    su = su_ref[...]
    sd = sd_ref[...]
    u = jnp.zeros((MB, D), jnp.float32)
    d = jnp.zeros((MB, D), jnp.float32)
    for k in range(PACK):
        u = u + jnp.where(su == k, u128_ref[:, k * D:(k + 1) * D], 0.0)
        d = d + jnp.where(sd == k, d128_ref[:, k * D:(k + 1) * D], 0.0)
    acc = (jnp.dot(u, w1u_ref[...], preferred_element_type=jnp.float32)
           + jnp.dot(d, w1d_ref[...], preferred_element_type=jnp.float32)
           + jnp.dot(img_ref[...], w1i_ref[...],
                     preferred_element_type=jnp.float32))
    h = jnp.maximum(acc + b1_ref[...], 0.0)
    out_ref[...] = jnp.sum(h * w2t_ref[...], axis=1) + b2_ref[0]


def kernel(user_idx, deal_idx, image_vec, user_table, deal_table, W1, b1, W2, b2):
    uidx = user_idx.astype(jnp.int32)
    didx = deal_idx.astype(jnp.int32)
    ugidx2d = (uidx // PACK).reshape(B // CHUNK, CHUNK)
    dgidx2d = (didx // PACK).reshape(B // CHUNK, CHUNK)
    utab = user_table.reshape(ROWS, PD)
    dtab = deal_table.reshape(ROWS, PD)
    u128, d128 = _sc_gather(ugidx2d, dgidx2d, utab, dtab)

    su2d = (uidx % PACK).reshape(B, 1)
    sd2d = (didx % PACK).reshape(B, 1)
    w1u, w1d, w1i = W1[:D], W1[D:2 * D], W1[2 * D:]
    b1r = b1.reshape(1, HIDDEN)
    w2t = W2.reshape(1, HIDDEN)

    score = pl.pallas_call(
        _mlp_body,
        grid=(B // MB,),
        in_specs=[
            pl.BlockSpec((MB, PD), lambda i: (i, 0)),
            pl.BlockSpec((MB, PD), lambda i: (i, 0)),
            pl.BlockSpec((MB, 1), lambda i: (i, 0)),
            pl.BlockSpec((MB, 1), lambda i: (i, 0)),
            pl.BlockSpec((MB, D), lambda i: (i, 0)),
            pl.BlockSpec((D, HIDDEN), lambda i: (0, 0)),
            pl.BlockSpec((D, HIDDEN), lambda i: (0, 0)),
            pl.BlockSpec((D, HIDDEN), lambda i: (0, 0)),
            pl.BlockSpec((1, HIDDEN), lambda i: (0, 0)),
            pl.BlockSpec((1, HIDDEN), lambda i: (0, 0)),
            pl.BlockSpec(memory_space=pltpu.SMEM),
        ],
        out_specs=pl.BlockSpec((MB,), lambda i: (i,)),
        out_shape=jax.ShapeDtypeStruct((B,), jnp.float32),
    )(u128, d128, su2d, sd2d, image_vec, w1u, w1d, w1i, b1r, w2t, b2)
    return score
